# baseline (speedup 1.0000x reference)
import functools

import jax
import jax.numpy as jnp
from jax.experimental import pallas as pl
from jax.experimental.pallas import tpu as pltpu


def _pass1_kernel(l_ref, xb_ref, yb_ref, lb_ref):
    lb = l_ref[...].astype(jnp.bfloat16)
    lb_ref[...] = lb
    y = jnp.dot(lb, xb_ref[...], preferred_element_type=jnp.float32)
    yb_ref[...] = y.astype(jnp.bfloat16)


def _pass2_kernel(lb_ref, xb_ref, xprev_ref, yb_ref):
    acc = jnp.dot(lb_ref[...], xb_ref[...], preferred_element_type=jnp.float32)
    yb_ref[...] = (2.0 * acc - xprev_ref[...]).astype(jnp.bfloat16)


def _pass3_kernel(lb_ref, x2b_ref, x0_ref, x1_ref, w_ref, b_ref,
                  out_ref, *, bm, f_out, pool):
    acc = jnp.dot(lb_ref[...], x2b_ref[...], preferred_element_type=jnp.float32)
    x3 = 2.0 * acc - x1_ref[...].astype(jnp.float32)
    i = pl.program_id(0)
    x0 = x0_ref[...].astype(jnp.bfloat16)
    x1 = x1_ref[...]
    x2 = x2b_ref[pl.ds(i * bm, bm), :]
    t = jnp.dot(x0, w_ref[0], preferred_element_type=jnp.float32)
    t = t + jnp.dot(x1, w_ref[1], preferred_element_type=jnp.float32)
    t = t + jnp.dot(x2, w_ref[2], preferred_element_type=jnp.float32)
    t = t + jnp.dot(x3.astype(jnp.bfloat16), w_ref[3],
                    preferred_element_type=jnp.float32)
    t = jnp.maximum(t + b_ref[...], 0.0)
    out_ref[...] = jnp.max(t.reshape(bm // pool, pool, f_out), axis=1)


_BM = 512


def kernel(x, laplacian, weight, bias):
    B, N, F_in = x.shape
    F_out = weight.shape[-1]
    degree = weight.shape[0] // F_in
    pool = 2
    bm = _BM
    nm = N // bm

    x0 = jnp.transpose(x, (1, 2, 0)).reshape(N, F_in * B)
    c = x0.shape[1]
    x0b = x0.astype(jnp.bfloat16)
    w4 = jnp.transpose(weight.reshape(F_in, degree, F_out), (1, 0, 2))
    w4 = w4.astype(jnp.bfloat16)
    b2 = bias.reshape(1, F_out)

    params = pltpu.CompilerParams(dimension_semantics=("arbitrary",))

    l_spec = pl.BlockSpec((bm, N), lambda i: (i, 0))
    vfull_spec = pl.BlockSpec((N, c), lambda i: (0, 0))
    vrow_spec = pl.BlockSpec((bm, c), lambda i: (i, 0))

    x1b, lb = pl.pallas_call(
        _pass1_kernel,
        grid=(nm,),
        in_specs=[l_spec, vfull_spec],
        out_specs=[vrow_spec, l_spec],
        out_shape=[
            jax.ShapeDtypeStruct((N, c), jnp.bfloat16),
            jax.ShapeDtypeStruct((N, N), jnp.bfloat16),
        ],
        compiler_params=params,
    )(laplacian, x0b)

    x2b = pl.pallas_call(
        _pass2_kernel,
        grid=(nm,),
        in_specs=[l_spec, vfull_spec, vrow_spec],
        out_specs=vrow_spec,
        out_shape=jax.ShapeDtypeStruct((N, c), jnp.bfloat16),
        compiler_params=params,
    )(lb, x1b, x0)

    out = pl.pallas_call(
        functools.partial(_pass3_kernel, bm=bm, f_out=F_out, pool=pool),
        grid=(nm,),
        in_specs=[
            l_spec,
            vfull_spec,
            vrow_spec,
            vrow_spec,
            pl.BlockSpec((degree, F_in, F_out), lambda i: (0, 0, 0)),
            pl.BlockSpec((1, F_out), lambda i: (0, 0)),
        ],
        out_specs=pl.BlockSpec((bm // pool, F_out), lambda i: (i, 0)),
        out_shape=jax.ShapeDtypeStruct((N // pool, F_out), jnp.float32),
        compiler_params=params,
    )(lb, x2b, x0, x1b, w4, b2)

    return out.reshape(B, N // pool, F_out)


# speedup vs baseline: 1.0099x; 1.0099x over previous
import functools

import jax
import jax.numpy as jnp
from jax import lax
from jax.experimental import pallas as pl
from jax.experimental.pallas import tpu as pltpu

_BM = 256


def _fused_kernel(l_hbm, x0b_ref, w_ref, b_ref, out_ref, lb_hbm,
                  x1b_ref, x2b_ref, lf_buf, sb_buf, lb_buf,
                  lf_sem, st_sem, lb_sem, *, bm, nm, f_out, pool):
    def load_f32(i, slot):
        return pltpu.make_async_copy(
            l_hbm.at[pl.ds(i * bm, bm), :], lf_buf.at[slot], lf_sem.at[slot])

    def store_b(i, slot):
        return pltpu.make_async_copy(
            sb_buf.at[slot], lb_hbm.at[pl.ds(i * bm, bm), :], st_sem.at[slot])

    def load_b(i, slot):
        return pltpu.make_async_copy(
            lb_hbm.at[pl.ds(i * bm, bm), :], lb_buf.at[slot], lb_sem.at[slot])

    load_f32(0, 0).start()
    load_f32(1, 1).start()

    def phase0(i, carry):
        slot = lax.rem(i, 2)
        load_f32(i, slot).wait()
        lb = lf_buf[slot].astype(jnp.bfloat16)

        @pl.when(i >= 2)
        def _():
            store_b(i - 2, slot).wait()

        sb_buf[slot] = lb
        store_b(i, slot).start()
        y = jnp.dot(lb, x0b_ref[...], preferred_element_type=jnp.float32)
        x1b_ref[pl.ds(i * bm, bm), :] = y.astype(jnp.bfloat16)

        @pl.when(i + 2 < nm)
        def _():
            load_f32(i + 2, slot).start()

        @pl.when(i + 2 >= nm)
        def _():
            load_b(i + 2 - nm, slot).start()

        return carry

    lax.fori_loop(0, nm, phase0, 0)
    store_b(nm - 2, 0).wait()
    store_b(nm - 1, 1).wait()

    def phase1(i, carry):
        slot = lax.rem(i, 2)
        load_b(i, slot).wait()
        z = jnp.dot(lb_buf[slot], x1b_ref[...],
                    preferred_element_type=jnp.float32)
        x0band = x0b_ref[pl.ds(i * bm, bm), :].astype(jnp.float32)
        x2b_ref[pl.ds(i * bm, bm), :] = (2.0 * z - x0band).astype(jnp.bfloat16)
        load_b(lax.rem(i + 2, nm), slot).start()
        return carry

    lax.fori_loop(0, nm, phase1, 0)

    def phase2(i, carry):
        slot = lax.rem(i, 2)
        load_b(i, slot).wait()
        z = jnp.dot(lb_buf[slot], x2b_ref[...],
                    preferred_element_type=jnp.float32)
        x1band = x1b_ref[pl.ds(i * bm, bm), :]
        x3 = 2.0 * z - x1band.astype(jnp.float32)
        t = jnp.dot(x0b_ref[pl.ds(i * bm, bm), :], w_ref[0],
                    preferred_element_type=jnp.float32)
        t = t + jnp.dot(x1band, w_ref[1], preferred_element_type=jnp.float32)
        t = t + jnp.dot(x2b_ref[pl.ds(i * bm, bm), :], w_ref[2],
                        preferred_element_type=jnp.float32)
        t = t + jnp.dot(x3.astype(jnp.bfloat16), w_ref[3],
                        preferred_element_type=jnp.float32)
        t = jnp.maximum(t + b_ref[...], 0.0)
        t = jnp.max(t.reshape(bm // pool, pool, f_out), axis=1)
        out_ref[pl.ds(i * (bm // pool), bm // pool), :] = t

        @pl.when(i + 2 < nm)
        def _():
            load_b(i + 2, slot).start()

        return carry

    lax.fori_loop(0, nm, phase2, 0)


def kernel(x, laplacian, weight, bias):
    B, N, F_in = x.shape
    F_out = weight.shape[-1]
    degree = weight.shape[0] // F_in
    pool = 2
    bm = _BM
    nm = N // bm

    x0 = jnp.transpose(x, (1, 2, 0)).reshape(N, F_in * B)
    c = x0.shape[1]
    x0b = x0.astype(jnp.bfloat16)
    w4 = jnp.transpose(weight.reshape(F_in, degree, F_out), (1, 0, 2))
    w4 = w4.astype(jnp.bfloat16)
    b2 = bias.reshape(1, F_out)

    out, _ = pl.pallas_call(
        functools.partial(_fused_kernel, bm=bm, nm=nm, f_out=F_out, pool=pool),
        in_specs=[
            pl.BlockSpec(memory_space=pltpu.MemorySpace.HBM),
            pl.BlockSpec(memory_space=pltpu.MemorySpace.VMEM),
            pl.BlockSpec(memory_space=pltpu.MemorySpace.VMEM),
            pl.BlockSpec(memory_space=pltpu.MemorySpace.VMEM),
        ],
        out_specs=[
            pl.BlockSpec(memory_space=pltpu.MemorySpace.VMEM),
            pl.BlockSpec(memory_space=pltpu.MemorySpace.HBM),
        ],
        out_shape=[
            jax.ShapeDtypeStruct((N // pool, F_out), jnp.float32),
            jax.ShapeDtypeStruct((N, N), jnp.bfloat16),
        ],
        scratch_shapes=[
            pltpu.VMEM((N, c), jnp.bfloat16),
            pltpu.VMEM((N, c), jnp.bfloat16),
            pltpu.VMEM((2, bm, N), jnp.float32),
            pltpu.VMEM((2, bm, N), jnp.bfloat16),
            pltpu.VMEM((2, bm, N), jnp.bfloat16),
            pltpu.SemaphoreType.DMA((2,)),
            pltpu.SemaphoreType.DMA((2,)),
            pltpu.SemaphoreType.DMA((2,)),
        ],
    )(laplacian, x0b, w4, b2)

    return out.reshape(B, N // pool, F_out)


# speedup vs baseline: 1.0281x; 1.0181x over previous
import functools

import jax
import jax.numpy as jnp
from jax import lax
from jax.experimental import pallas as pl
from jax.experimental.pallas import tpu as pltpu

_BM0 = 128
_BM = 128
_NSLOT = 4
_KR = 2048


def _fused_kernel(l_hbm, x0b_ref, w_ref, b_ref, out_ref, lb_hbm,
                  x1b_ref, x2b_ref, lbr_ref, lf_buf, sb_buf, lb_buf,
                  lf_sem, st_sem, lb_sem, *, n, kr, bm0, bm, f_out, pool):
    nm0 = n // bm0
    nr0 = kr // bm0
    nm = n // bm
    nr = kr // bm
    ns = nm - nr

    def load_f32(i, slot):
        return pltpu.make_async_copy(
            l_hbm.at[pl.ds(i * bm0, bm0), :], lf_buf.at[slot],
            lf_sem.at[slot])

    def store_b(i, slot):
        return pltpu.make_async_copy(
            sb_buf.at[slot], lb_hbm.at[pl.ds(i * bm0 - kr, bm0), :],
            st_sem.at[slot])

    def load_b(j, slot):
        return pltpu.make_async_copy(
            lb_hbm.at[pl.ds(j * bm, bm), :], lb_buf.at[slot],
            lb_sem.at[slot])

    load_f32(0, 0).start()
    load_f32(1, 1).start()

    def p0_step(i, lband):
        y = jnp.dot(lband, x0b_ref[...], preferred_element_type=jnp.float32)
        x1b_ref[pl.ds(i * bm0, bm0), :] = y.astype(jnp.bfloat16)

    def phase0_res(i, carry):
        slot = lax.rem(i, 2)
        load_f32(i, slot).wait()
        lbr_ref[pl.ds(i * bm0, bm0), :] = lf_buf[slot].astype(jnp.bfloat16)
        p0_step(i, lbr_ref[pl.ds(i * bm0, bm0), :])
        load_f32(i + 2, slot).start()
        return carry

    lax.fori_loop(0, nr0, phase0_res, 0)

    def phase0_str(i, carry):
        slot = lax.rem(i, 2)
        load_f32(i, slot).wait()

        @pl.when(i >= nr0 + 2)
        def _():
            store_b(i - 2, slot).wait()

        sb_buf[slot] = lf_buf[slot].astype(jnp.bfloat16)
        store_b(i, slot).start()
        p0_step(i, sb_buf[slot])

        @pl.when(i + 2 < nm0)
        def _():
            load_f32(i + 2, slot).start()

        @pl.when(i >= nm0 - 4)
        def _():
            load_b(i - (nm0 - 4), lax.rem(i - (nm0 - 4), _NSLOT)).start()

        return carry

    lax.fori_loop(nr0, nm0, phase0_str, 0)
    store_b(nm0 - 2, 0).wait()
    store_b(nm0 - 1, 1).wait()

    def p1_step(j, lband):
        z = jnp.dot(lband, x1b_ref[...], preferred_element_type=jnp.float32)
        x0band = x0b_ref[pl.ds(j * bm, bm), :].astype(jnp.float32)
        x2b_ref[pl.ds(j * bm, bm), :] = (2.0 * z - x0band).astype(jnp.bfloat16)

    def phase1_str(js, carry):
        slot = lax.rem(js, _NSLOT)
        load_b(js, slot).wait()
        p1_step(nr + js, lb_buf[slot])
        load_b(lax.rem(js + 4, ns), slot).start()
        return carry

    lax.fori_loop(0, ns, phase1_str, 0)

    def phase1_res(j, carry):
        p1_step(j, lbr_ref[pl.ds(j * bm, bm), :])
        return carry

    lax.fori_loop(0, nr, phase1_res, 0)

    def p2_step(j, lband):
        z = jnp.dot(lband, x2b_ref[...], preferred_element_type=jnp.float32)
        x1band = x1b_ref[pl.ds(j * bm, bm), :]
        x3 = 2.0 * z - x1band.astype(jnp.float32)
        t = jnp.dot(x0b_ref[pl.ds(j * bm, bm), :], w_ref[0],
                    preferred_element_type=jnp.float32)
        t = t + jnp.dot(x1band, w_ref[1], preferred_element_type=jnp.float32)
        t = t + jnp.dot(x2b_ref[pl.ds(j * bm, bm), :], w_ref[2],
                        preferred_element_type=jnp.float32)
        t = t + jnp.dot(x3.astype(jnp.bfloat16), w_ref[3],
                        preferred_element_type=jnp.float32)
        t = jnp.maximum(t + b_ref[...], 0.0)
        t = jnp.max(t.reshape(bm // pool, pool, f_out), axis=1)
        out_ref[pl.ds(j * (bm // pool), bm // pool), :] = t

    def phase2_str(js, carry):
        slot = lax.rem(js, _NSLOT)
        load_b(js, slot).wait()
        p2_step(nr + js, lb_buf[slot])

        @pl.when(js + 4 < ns)
        def _():
            load_b(js + 4, slot).start()

        return carry

    lax.fori_loop(0, ns, phase2_str, 0)

    def phase2_res(j, carry):
        p2_step(j, lbr_ref[pl.ds(j * bm, bm), :])
        return carry

    lax.fori_loop(0, nr, phase2_res, 0)


def kernel(x, laplacian, weight, bias):
    B, N, F_in = x.shape
    F_out = weight.shape[-1]
    degree = weight.shape[0] // F_in
    pool = 2

    x0 = jnp.transpose(x, (1, 2, 0)).reshape(N, F_in * B)
    c = x0.shape[1]
    x0b = x0.astype(jnp.bfloat16)
    w4 = jnp.transpose(weight.reshape(F_in, degree, F_out), (1, 0, 2))
    w4 = w4.astype(jnp.bfloat16)
    b2 = bias.reshape(1, F_out)

    out, _ = pl.pallas_call(
        functools.partial(_fused_kernel, n=N, kr=_KR, bm0=_BM0, bm=_BM,
                          f_out=F_out, pool=pool),
        compiler_params=pltpu.CompilerParams(
            vmem_limit_bytes=110 * 1024 * 1024),
        in_specs=[
            pl.BlockSpec(memory_space=pltpu.MemorySpace.HBM),
            pl.BlockSpec(memory_space=pltpu.MemorySpace.VMEM),
            pl.BlockSpec(memory_space=pltpu.MemorySpace.VMEM),
            pl.BlockSpec(memory_space=pltpu.MemorySpace.VMEM),
        ],
        out_specs=[
            pl.BlockSpec(memory_space=pltpu.MemorySpace.VMEM),
            pl.BlockSpec(memory_space=pltpu.MemorySpace.HBM),
        ],
        out_shape=[
            jax.ShapeDtypeStruct((N // pool, F_out), jnp.float32),
            jax.ShapeDtypeStruct((N - _KR, N), jnp.bfloat16),
        ],
        scratch_shapes=[
            pltpu.VMEM((N, c), jnp.bfloat16),
            pltpu.VMEM((N, c), jnp.bfloat16),
            pltpu.VMEM((_KR, N), jnp.bfloat16),
            pltpu.VMEM((2, _BM0, N), jnp.float32),
            pltpu.VMEM((2, _BM0, N), jnp.bfloat16),
            pltpu.VMEM((_NSLOT, _BM, N), jnp.bfloat16),
            pltpu.SemaphoreType.DMA((2,)),
            pltpu.SemaphoreType.DMA((2,)),
            pltpu.SemaphoreType.DMA((_NSLOT,)),
        ],
    )(laplacian, x0b, w4, b2)

    return out.reshape(B, N // pool, F_out)


# speedup vs baseline: 1.0910x; 1.0611x over previous
import functools

import jax
import jax.numpy as jnp
from jax import lax
from jax.experimental import pallas as pl
from jax.experimental.pallas import tpu as pltpu

_BM0 = 128
_BM = 256
_NSLOT = 2
_KR = 2048


def _fused_kernel(l_hbm, x0b_ref, w_ref, b_ref, out_ref, lb_hbm,
                  x1b_ref, x2b_ref, lbr_ref, lf_buf, sb_buf, lb_buf,
                  lf_sem, st_sem, lb_sem, *, n, kr, bm0, bm, f_out, pool):
    nm0 = n // bm0
    nr0 = kr // bm0
    nm = n // bm
    nr = kr // bm
    ns = nm - nr

    def load_f32(i, slot):
        return pltpu.make_async_copy(
            l_hbm.at[pl.ds(i * bm0, bm0), :], lf_buf.at[slot],
            lf_sem.at[slot])

    def store_b(i, slot):
        return pltpu.make_async_copy(
            sb_buf.at[slot], lb_hbm.at[pl.ds(i * bm0 - kr, bm0), :],
            st_sem.at[slot])

    def load_b(j, slot):
        return pltpu.make_async_copy(
            lb_hbm.at[pl.ds(j * bm, bm), :], lb_buf.at[slot],
            lb_sem.at[slot])

    load_f32(0, 0).start()
    load_f32(1, 1).start()

    def p0_step(i, lband):
        y = jnp.dot(lband, x0b_ref[...], preferred_element_type=jnp.float32)
        x1b_ref[pl.ds(i * bm0, bm0), :] = y.astype(jnp.bfloat16)

    def phase0_res(i, carry):
        slot = lax.rem(i, 2)
        load_f32(i, slot).wait()
        lbr_ref[pl.ds(i * bm0, bm0), :] = lf_buf[slot].astype(jnp.bfloat16)
        p0_step(i, lbr_ref[pl.ds(i * bm0, bm0), :])
        load_f32(i + 2, slot).start()
        return carry

    lax.fori_loop(0, nr0, phase0_res, 0)

    def phase0_str(i, carry):
        slot = lax.rem(i, 2)
        load_f32(i, slot).wait()

        @pl.when(i >= nr0 + 2)
        def _():
            store_b(i - 2, slot).wait()

        sb_buf[slot] = lf_buf[slot].astype(jnp.bfloat16)
        store_b(i, slot).start()
        p0_step(i, sb_buf[slot])

        @pl.when(i + 2 < nm0)
        def _():
            load_f32(i + 2, slot).start()

        @pl.when(i >= nm0 - _NSLOT)
        def _():
            load_b(i - (nm0 - _NSLOT), lax.rem(i - (nm0 - _NSLOT), _NSLOT)).start()

        return carry

    lax.fori_loop(nr0, nm0, phase0_str, 0)
    store_b(nm0 - 2, 0).wait()
    store_b(nm0 - 1, 1).wait()

    def p1_step(j, lband):
        z = jnp.dot(lband, x1b_ref[...], preferred_element_type=jnp.float32)
        x0band = x0b_ref[pl.ds(j * bm, bm), :].astype(jnp.float32)
        x2b_ref[pl.ds(j * bm, bm), :] = (2.0 * z - x0band).astype(jnp.bfloat16)

    def phase1_str(js, carry):
        slot = lax.rem(js, _NSLOT)
        load_b(js, slot).wait()
        p1_step(nr + js, lb_buf[slot])
        load_b(lax.rem(js + _NSLOT, ns), slot).start()
        return carry

    lax.fori_loop(0, ns, phase1_str, 0)

    def phase1_res(j, carry):
        p1_step(j, lbr_ref[pl.ds(j * bm, bm), :])
        return carry

    lax.fori_loop(0, nr, phase1_res, 0)

    def p2_step(j, lband):
        z = jnp.dot(lband, x2b_ref[...], preferred_element_type=jnp.float32)
        x1band = x1b_ref[pl.ds(j * bm, bm), :]
        x3 = 2.0 * z - x1band.astype(jnp.float32)
        t = jnp.dot(x0b_ref[pl.ds(j * bm, bm), :], w_ref[0],
                    preferred_element_type=jnp.float32)
        t = t + jnp.dot(x1band, w_ref[1], preferred_element_type=jnp.float32)
        t = t + jnp.dot(x2b_ref[pl.ds(j * bm, bm), :], w_ref[2],
                        preferred_element_type=jnp.float32)
        t = t + jnp.dot(x3.astype(jnp.bfloat16), w_ref[3],
                        preferred_element_type=jnp.float32)
        t = jnp.maximum(t + b_ref[...], 0.0)
        t = jnp.max(t.reshape(bm // pool, pool, f_out), axis=1)
        out_ref[pl.ds(j * (bm // pool), bm // pool), :] = t

    def phase2_str(js, carry):
        slot = lax.rem(js, _NSLOT)
        load_b(js, slot).wait()
        p2_step(nr + js, lb_buf[slot])

        @pl.when(js + _NSLOT < ns)
        def _():
            load_b(js + _NSLOT, slot).start()

        return carry

    lax.fori_loop(0, ns, phase2_str, 0)

    def phase2_res(j, carry):
        p2_step(j, lbr_ref[pl.ds(j * bm, bm), :])
        return carry

    lax.fori_loop(0, nr, phase2_res, 0)


def kernel(x, laplacian, weight, bias):
    B, N, F_in = x.shape
    F_out = weight.shape[-1]
    degree = weight.shape[0] // F_in
    pool = 2

    x0 = jnp.transpose(x, (1, 2, 0)).reshape(N, F_in * B)
    c = x0.shape[1]
    x0b = x0.astype(jnp.bfloat16)
    w4 = jnp.transpose(weight.reshape(F_in, degree, F_out), (1, 0, 2))
    w4 = w4.astype(jnp.bfloat16)
    b2 = bias.reshape(1, F_out)

    out, _ = pl.pallas_call(
        functools.partial(_fused_kernel, n=N, kr=_KR, bm0=_BM0, bm=_BM,
                          f_out=F_out, pool=pool),
        compiler_params=pltpu.CompilerParams(
            vmem_limit_bytes=110 * 1024 * 1024),
        in_specs=[
            pl.BlockSpec(memory_space=pltpu.MemorySpace.HBM),
            pl.BlockSpec(memory_space=pltpu.MemorySpace.VMEM),
            pl.BlockSpec(memory_space=pltpu.MemorySpace.VMEM),
            pl.BlockSpec(memory_space=pltpu.MemorySpace.VMEM),
        ],
        out_specs=[
            pl.BlockSpec(memory_space=pltpu.MemorySpace.VMEM),
            pl.BlockSpec(memory_space=pltpu.MemorySpace.HBM),
        ],
        out_shape=[
            jax.ShapeDtypeStruct((N // pool, F_out), jnp.float32),
            jax.ShapeDtypeStruct((N - _KR, N), jnp.bfloat16),
        ],
        scratch_shapes=[
            pltpu.VMEM((N, c), jnp.bfloat16),
            pltpu.VMEM((N, c), jnp.bfloat16),
            pltpu.VMEM((_KR, N), jnp.bfloat16),
            pltpu.VMEM((2, _BM0, N), jnp.float32),
            pltpu.VMEM((2, _BM0, N), jnp.bfloat16),
            pltpu.VMEM((_NSLOT, _BM, N), jnp.bfloat16),
            pltpu.SemaphoreType.DMA((2,)),
            pltpu.SemaphoreType.DMA((2,)),
            pltpu.SemaphoreType.DMA((_NSLOT,)),
        ],
    )(laplacian, x0b, w4, b2)

    return out.reshape(B, N // pool, F_out)


# speedup vs baseline: 1.1079x; 1.0156x over previous
import functools

import jax
import jax.numpy as jnp
from jax import lax
from jax.experimental import pallas as pl
from jax.experimental.pallas import tpu as pltpu

_BM0 = 128
_BM = 256
_NSLOT = 2
_KR = 2048


def _fused_kernel(l_hbm, x0b_ref, w_ref, b_ref, out_ref, lb_hbm,
                  x1b_ref, x2b_ref, lbr_ref, lf_buf, sb_buf, lb_buf,
                  lf_sem, st_sem, lb_sem, *, n, kr, bm0, bm, f_out, pool):
    nm0 = n // bm0
    nr0 = kr // bm0
    nm = n // bm
    nr = kr // bm
    ns = nm - nr

    def load_f32(i, slot):
        return pltpu.make_async_copy(
            l_hbm.at[pl.ds(i * bm0, bm0), :], lf_buf.at[slot],
            lf_sem.at[slot])

    def store_b(i, slot):
        return pltpu.make_async_copy(
            sb_buf.at[slot], lb_hbm.at[pl.ds(i * bm0 - kr, bm0), :],
            st_sem.at[slot])

    def load_b(j, slot):
        return pltpu.make_async_copy(
            lb_hbm.at[pl.ds(j * bm, bm), :], lb_buf.at[slot],
            lb_sem.at[slot])

    load_f32(0, 0).start()
    load_f32(1, 1).start()

    def p0_step(i, lband):
        y = jnp.dot(lband, x0b_ref[...], preferred_element_type=jnp.float32)
        x1b_ref[pl.ds(i * bm0, bm0), :] = y.astype(jnp.bfloat16)

    def phase0_res(i, carry):
        slot = lax.rem(i, 2)
        load_f32(i, slot).wait()
        lbr_ref[pl.ds(i * bm0, bm0), :] = lf_buf[slot].astype(jnp.bfloat16)
        p0_step(i, lbr_ref[pl.ds(i * bm0, bm0), :])
        load_f32(i + 2, slot).start()
        return carry

    lax.fori_loop(0, nr0, phase0_res, 0)

    def phase0_str(i, carry):
        slot = lax.rem(i, 2)
        load_f32(i, slot).wait()

        @pl.when(i >= nr0 + 2)
        def _():
            store_b(i - 2, slot).wait()

        sb_buf[slot] = lf_buf[slot].astype(jnp.bfloat16)
        store_b(i, slot).start()
        p0_step(i, sb_buf[slot])

        @pl.when(i + 2 < nm0)
        def _():
            load_f32(i + 2, slot).start()

        @pl.when(i >= nm0 - _NSLOT)
        def _():
            load_b(i - (nm0 - _NSLOT), lax.rem(i - (nm0 - _NSLOT), _NSLOT)).start()

        return carry

    lax.fori_loop(nr0, nm0, phase0_str, 0)
    store_b(nm0 - 2, 0).wait()
    store_b(nm0 - 1, 1).wait()

    def p1_step(j, lband):
        z = jnp.dot(lband, x1b_ref[...], preferred_element_type=jnp.float32)
        x0band = x0b_ref[pl.ds(j * bm, bm), :].astype(jnp.float32)
        x2b_ref[pl.ds(j * bm, bm), :] = (2.0 * z - x0band).astype(jnp.bfloat16)

    rat = ns // nr

    def phase1_str(js, carry):
        slot = lax.rem(js, _NSLOT)
        load_b(js, slot).wait()
        p1_step(nr + js, lb_buf[slot])
        load_b(lax.rem(js + _NSLOT, ns), slot).start()

        @pl.when(lax.rem(js, rat) == 0)
        def _():
            jr = lax.div(js, rat)
            p1_step(jr, lbr_ref[pl.ds(jr * bm, bm), :])

        return carry

    lax.fori_loop(0, ns, phase1_str, 0)

    def p2_step(j, lband):
        z = jnp.dot(lband, x2b_ref[...], preferred_element_type=jnp.float32)
        x1band = x1b_ref[pl.ds(j * bm, bm), :]
        x3 = 2.0 * z - x1band.astype(jnp.float32)
        t = jnp.dot(x0b_ref[pl.ds(j * bm, bm), :], w_ref[0],
                    preferred_element_type=jnp.float32)
        t = t + jnp.dot(x1band, w_ref[1], preferred_element_type=jnp.float32)
        t = t + jnp.dot(x2b_ref[pl.ds(j * bm, bm), :], w_ref[2],
                        preferred_element_type=jnp.float32)
        t = t + jnp.dot(x3.astype(jnp.bfloat16), w_ref[3],
                        preferred_element_type=jnp.float32)
        t = jnp.maximum(t + b_ref[...], 0.0)
        t = jnp.max(t.reshape(bm // pool, pool, f_out), axis=1)
        out_ref[pl.ds(j * (bm // pool), bm // pool), :] = t

    def phase2_str(js, carry):
        slot = lax.rem(js, _NSLOT)
        load_b(js, slot).wait()
        p2_step(nr + js, lb_buf[slot])

        @pl.when(js + _NSLOT < ns)
        def _():
            load_b(js + _NSLOT, slot).start()

        @pl.when(lax.rem(js, rat) == 0)
        def _():
            jr = lax.div(js, rat)
            p2_step(jr, lbr_ref[pl.ds(jr * bm, bm), :])

        return carry

    lax.fori_loop(0, ns, phase2_str, 0)


def kernel(x, laplacian, weight, bias):
    B, N, F_in = x.shape
    F_out = weight.shape[-1]
    degree = weight.shape[0] // F_in
    pool = 2

    x0 = jnp.transpose(x, (1, 2, 0)).reshape(N, F_in * B)
    c = x0.shape[1]
    x0b = x0.astype(jnp.bfloat16)
    w4 = jnp.transpose(weight.reshape(F_in, degree, F_out), (1, 0, 2))
    w4 = w4.astype(jnp.bfloat16)
    b2 = bias.reshape(1, F_out)

    out, _ = pl.pallas_call(
        functools.partial(_fused_kernel, n=N, kr=_KR, bm0=_BM0, bm=_BM,
                          f_out=F_out, pool=pool),
        compiler_params=pltpu.CompilerParams(
            vmem_limit_bytes=110 * 1024 * 1024),
        in_specs=[
            pl.BlockSpec(memory_space=pltpu.MemorySpace.HBM),
            pl.BlockSpec(memory_space=pltpu.MemorySpace.VMEM),
            pl.BlockSpec(memory_space=pltpu.MemorySpace.VMEM),
            pl.BlockSpec(memory_space=pltpu.MemorySpace.VMEM),
        ],
        out_specs=[
            pl.BlockSpec(memory_space=pltpu.MemorySpace.VMEM),
            pl.BlockSpec(memory_space=pltpu.MemorySpace.HBM),
        ],
        out_shape=[
            jax.ShapeDtypeStruct((N // pool, F_out), jnp.float32),
            jax.ShapeDtypeStruct((N - _KR, N), jnp.bfloat16),
        ],
        scratch_shapes=[
            pltpu.VMEM((N, c), jnp.bfloat16),
            pltpu.VMEM((N, c), jnp.bfloat16),
            pltpu.VMEM((_KR, N), jnp.bfloat16),
            pltpu.VMEM((2, _BM0, N), jnp.float32),
            pltpu.VMEM((2, _BM0, N), jnp.bfloat16),
            pltpu.VMEM((_NSLOT, _BM, N), jnp.bfloat16),
            pltpu.SemaphoreType.DMA((2,)),
            pltpu.SemaphoreType.DMA((2,)),
            pltpu.SemaphoreType.DMA((_NSLOT,)),
        ],
    )(laplacian, x0b, w4, b2)

    return out.reshape(B, N // pool, F_out)
